# baseline (speedup 1.0000x reference)
import functools

import jax
import jax.numpy as jnp
from jax import lax
from jax.experimental import pallas as pl
from jax.experimental.pallas import tpu as pltpu
from jax.experimental.pallas import tpu_sc as plsc

NC = 2
NS = 16
L = 16
CHUNK = 80


def _sc_spmm(x, src, dst, vals, zrows):
    n, d = x.shape
    e = vals.shape[0]
    edges_per_tile = e // (NC * NS)
    nchunk = edges_per_tile // CHUNK
    r0 = (n // NS) // 8 * 8
    r_last = n - (NS - 1) * r0

    mesh = plsc.VectorSubcoreMesh(core_axis_name="c", subcore_axis_name="s")

    @functools.partial(
        pl.kernel,
        mesh=mesh,
        out_type=jax.ShapeDtypeStruct((NC * n, d), jnp.float32),
        scratch_types=[
            pltpu.VMEM((4, CHUNK), jnp.int32),
            pltpu.VMEM((4, CHUNK), jnp.int32),
            pltpu.VMEM((2, CHUNK), jnp.float32),
            pltpu.VMEM((2, CHUNK, d), jnp.float32),
            pltpu.VMEM((2, CHUNK, d), jnp.float32),
            pltpu.VMEM_SHARED((n, d), jnp.float32),
        ] + [pltpu.SemaphoreType.DMA] * 10,
    )
    def spmm(x_hbm, src_hbm, dst_hbm, vals_hbm, z_hbm, out_hbm,
             srcr, dstr, valr, gbuf, sbuf, acc_sh,
             sg0, sg1, ss0, ss1, si0, si1, sd0, sd1, sv0, sv1):
        sems_g = (sg0, sg1)
        sems_s = (ss0, ss1)
        sems_i = (si0, si1)
        sems_d = (sd0, sd1)
        sems_v = (sv0, sv1)

        cid = lax.axis_index("c")
        sid = lax.axis_index("s")
        wid = cid * NS + sid

        rbase = sid * r0

        @pl.when(sid < NS - 1)
        def _zero_main():
            pltpu.sync_copy(z_hbm.at[pl.ds(0, r0)],
                            acc_sh.at[pl.ds(rbase, r0)])

        @pl.when(sid == NS - 1)
        def _zero_last():
            pltpu.sync_copy(z_hbm.at[pl.ds(0, r_last)],
                            acc_sh.at[pl.ds((NS - 1) * r0, r_last)])

        ebase = wid * edges_per_tile

        def src_fetch(c, slot, sem):
            return pltpu.make_async_copy(
                src_hbm.at[pl.ds(ebase + c * CHUNK, CHUNK)], srcr.at[slot], sem)

        def dst_fetch(c, slot, sem):
            return pltpu.make_async_copy(
                dst_hbm.at[pl.ds(ebase + c * CHUNK, CHUNK)], dstr.at[slot], sem)

        def val_fetch(c, b, sem):
            return pltpu.make_async_copy(
                vals_hbm.at[pl.ds(ebase + c * CHUNK, CHUNK)], valr.at[b], sem)

        def gather(slot, b, sem):
            return pltpu.make_async_copy(
                x_hbm.at[srcr.at[slot]], gbuf.at[b], sem)

        def scatter_start(slot, b, sem):
            pltpu.async_copy(sbuf.at[b], acc_sh.at[dstr.at[slot]], sem,
                             add=True)

        def scatter_wait(slot, b, sem):
            pltpu.make_async_copy(
                sbuf.at[b], acc_sh.at[dstr.at[slot]], sem).wait()

        plsc.subcore_barrier()

        for k in range(2):
            pltpu.sync_copy(src_hbm.at[pl.ds(ebase + k * CHUNK, CHUNK)],
                            srcr.at[k])
            gather(k, k, sems_g[k]).start()
            src_fetch(2 + k, 2 + k, sems_i[k]).start()
            dst_fetch(k, k, sems_d[k]).start()
            val_fetch(k, k, sems_v[k]).start()

        def step(i, carry):
            for b in range(2):
                c = i * 2 + b

                @pl.when(c < nchunk)
                def _process():
                    s_c = lax.rem(c, 4)
                    gather(s_c, b, sems_g[b]).wait()
                    dst_fetch(c, s_c, sems_d[b]).wait()
                    val_fetch(c, b, sems_v[b]).wait()

                    @pl.when(c >= 2)
                    def _drain_old_scatter():
                        scatter_wait(lax.rem(c + 2, 4), b, sems_s[b])

                    def group_body(g, carry2):
                        vals16 = valr[b, pl.ds(g * L, L)]
                        for lane in range(L):
                            vv = jnp.broadcast_to(vals16[lane], (L,))
                            k = g * L + lane
                            for j in range(d // L):
                                s = pl.ds(j * L, L)
                                sbuf[b, k, s] = gbuf[b, k, s] * vv
                        return carry2

                    lax.fori_loop(0, CHUNK // L, group_body, 0)

                    scatter_start(s_c, b, sems_s[b])

                    @pl.when(c + 2 < nchunk)
                    def _prefetch_next():
                        src_fetch(c + 2, lax.rem(c + 2, 4), sems_i[b]).wait()
                        gather(lax.rem(c + 2, 4), b, sems_g[b]).start()
                        dst_fetch(c + 2, lax.rem(c + 2, 4), sems_d[b]).start()
                        val_fetch(c + 2, b, sems_v[b]).start()

                    @pl.when(c + 4 < nchunk)
                    def _prefetch_src():
                        src_fetch(c + 4, s_c, sems_i[b]).start()

            return carry

        lax.fori_loop(0, (nchunk + 1) // 2, step, 0)

        for c in (nchunk - 2, nchunk - 1):
            scatter_wait(c % 4, c % 2, sems_s[c % 2])

        plsc.subcore_barrier()
        plsc.subcore_barrier()

        @pl.when(sid < NS - 1)
        def _write_main():
            pltpu.sync_copy(acc_sh.at[pl.ds(rbase, r0)],
                            out_hbm.at[pl.ds(cid * n + rbase, r0)])

        @pl.when(sid == NS - 1)
        def _write_last():
            pltpu.sync_copy(
                acc_sh.at[pl.ds((NS - 1) * r0, r_last)],
                out_hbm.at[pl.ds(cid * n + (NS - 1) * r0, r_last)])

    return spmm(x, src, dst, vals, zrows)


def _tc_combine_matmul(p0, p1, W, b2d):
    n, d_in = p0.shape
    d_out = W.shape[1]
    bm = 1000

    def body(p0_ref, p1_ref, w_ref, b_ref, o_ref):
        acc = p0_ref[...] + p1_ref[...]
        o_ref[...] = (
            jnp.dot(acc, w_ref[...], preferred_element_type=jnp.float32)
            + b_ref[...])

    return pl.pallas_call(
        body,
        grid=(n // bm,),
        in_specs=[
            pl.BlockSpec((bm, d_in), lambda i: (i, 0)),
            pl.BlockSpec((bm, d_in), lambda i: (i, 0)),
            pl.BlockSpec((d_in, d_out), lambda i: (0, 0)),
            pl.BlockSpec((1, d_out), lambda i: (0, 0)),
        ],
        out_specs=pl.BlockSpec((bm, d_out), lambda i: (i, 0)),
        out_shape=jax.ShapeDtypeStruct((n, d_out), jnp.float32),
    )(p0, p1, W, b2d)


def kernel(x, edge_index, edge_vals, W, b):
    n, d = x.shape
    e = edge_vals.shape[0]
    assert e % (NC * NS * CHUNK) == 0 and n % NS == 0 and d % L == 0

    src = edge_index[1]
    dst = edge_index[0]
    r_last = n - (NS - 1) * ((n // NS) // 8 * 8)
    zrows = jnp.zeros((r_last, d), jnp.float32)

    partials = _sc_spmm(x, src, dst, edge_vals, zrows)
    return _tc_combine_matmul(partials[:n], partials[n:], W,
                              b.reshape(1, -1))


# speedup vs baseline: 1.0494x; 1.0494x over previous
import functools

import jax
import jax.numpy as jnp
from jax import lax
from jax.experimental import pallas as pl
from jax.experimental.pallas import tpu as pltpu
from jax.experimental.pallas import tpu_sc as plsc

NC = 2
NS = 16
L = 16
CHUNK = 80


def _sc_spmm(x, src, dst, vals, zrows):
    n, d = x.shape
    e = vals.shape[0]
    edges_per_tile = e // (NC * NS)
    nchunk = edges_per_tile // CHUNK
    r0 = (n // NS) // 8 * 8
    r_last = n - (NS - 1) * r0

    mesh = plsc.VectorSubcoreMesh(core_axis_name="c", subcore_axis_name="s")

    GF = 5
    ngroup = nchunk // GF
    RR = 2 * GF

    @functools.partial(
        pl.kernel,
        mesh=mesh,
        out_type=jax.ShapeDtypeStruct((NC * n, d), jnp.float32),
        scratch_types=[
            pltpu.VMEM((RR * CHUNK,), jnp.int32),
            pltpu.VMEM((RR, CHUNK), jnp.int32),
            pltpu.VMEM((RR * CHUNK,), jnp.float32),
            pltpu.VMEM((2, CHUNK, d), jnp.float32),
            pltpu.VMEM((2, CHUNK, d), jnp.float32),
            pltpu.VMEM_SHARED((n, d), jnp.float32),
        ] + [pltpu.SemaphoreType.DMA] * 7,
    )
    def spmm(x_hbm, src_hbm, dst_hbm, vals_hbm, z_hbm, out_hbm,
             srcr, dstr, valr, gbuf, sbuf, acc_sh,
             sg0, sg1, ss0, ss1, sem_i, sem_d, sem_v):
        sems_g = (sg0, sg1)
        sems_s = (ss0, ss1)

        cid = lax.axis_index("c")
        sid = lax.axis_index("s")
        wid = cid * NS + sid
        ebase = wid * edges_per_tile
        gbase = wid * ngroup

        def src_group_fetch(g):
            return pltpu.make_async_copy(
                src_hbm.at[pl.ds(ebase + g * GF * CHUNK, GF * CHUNK)],
                srcr.at[pl.ds(lax.rem(g, 2) * GF * CHUNK, GF * CHUNK)],
                sem_i)

        def dst_group_fetch(g):
            return pltpu.make_async_copy(
                dst_hbm.at[gbase + g],
                dstr.at[pl.ds(lax.rem(g, 2) * GF, GF)],
                sem_d)

        def val_group_fetch(g):
            return pltpu.make_async_copy(
                vals_hbm.at[pl.ds(ebase + g * GF * CHUNK, GF * CHUNK)],
                valr.at[pl.ds(lax.rem(g, 2) * GF * CHUNK, GF * CHUNK)],
                sem_v)

        def gather(c, b, sem):
            idx = srcr.at[pl.ds(lax.rem(c, RR) * CHUNK, CHUNK)]
            return pltpu.make_async_copy(x_hbm.at[idx], gbuf.at[b], sem)

        def scatter_start(c, b, sem):
            pltpu.async_copy(sbuf.at[b], acc_sh.at[dstr.at[lax.rem(c, RR)]],
                             sem, add=True)

        def scatter_wait(c, b, sem):
            pltpu.make_async_copy(
                sbuf.at[b], acc_sh.at[dstr.at[lax.rem(c, RR)]], sem).wait()

        pltpu.sync_copy(src_hbm.at[pl.ds(ebase, GF * CHUNK)],
                        srcr.at[pl.ds(0, GF * CHUNK)])
        gather(0, 0, sems_g[0]).start()
        gather(1, 1, sems_g[1]).start()
        src_group_fetch(1).start()
        dst_group_fetch(0).start()
        val_group_fetch(0).start()

        rbase = sid * r0

        @pl.when(sid < NS - 1)
        def _zero_main():
            pltpu.sync_copy(z_hbm.at[pl.ds(0, r0)],
                            acc_sh.at[pl.ds(rbase, r0)])

        @pl.when(sid == NS - 1)
        def _zero_last():
            pltpu.sync_copy(z_hbm.at[pl.ds(0, r_last)],
                            acc_sh.at[pl.ds((NS - 1) * r0, r_last)])

        plsc.subcore_barrier()

        def step(i, carry):
            for b in range(2):
                c = i * 2 + b
                cg = lax.rem(c, GF)
                g = lax.div(c, GF)

                @pl.when(c < nchunk)
                def _process():
                    @pl.when(cg == 0)
                    def _enter_group():
                        dst_group_fetch(g).wait()
                        val_group_fetch(g).wait()

                        @pl.when(c + GF < nchunk)
                        def _issue_next_vals():
                            val_group_fetch(g + 1).start()
                            src_group_fetch(g + 1).start()

                    gather(c, b, sems_g[b]).wait()

                    @pl.when(c >= 2)
                    def _drain_old_scatter():
                        scatter_wait(c - 2, b, sems_s[b])

                    @pl.when(cg == 2)
                    def _issue_next_dst():
                        @pl.when(c + GF - 2 < nchunk)
                        def _issue():
                            dst_group_fetch(g + 1).start()

                    vb = lax.rem(c, RR) * CHUNK

                    def group_body(gg, carry2):
                        vals16 = valr[pl.ds(vb + gg * L, L)]
                        for lane in range(L):
                            vv = jnp.broadcast_to(vals16[lane], (L,))
                            k = gg * L + lane
                            for j in range(d // L):
                                s = pl.ds(j * L, L)
                                sbuf[b, k, s] = gbuf[b, k, s] * vv
                        return carry2

                    lax.fori_loop(0, CHUNK // L, group_body, 0)

                    scatter_start(c, b, sems_s[b])

                    @pl.when(c + 2 < nchunk)
                    def _prefetch_next():
                        @pl.when(lax.rem(c + 2, GF) == 0)
                        def _wait_next_src_group():
                            src_group_fetch(g + 1).wait()

                        gather(c + 2, b, sems_g[b]).start()

            return carry

        lax.fori_loop(0, (nchunk + 1) // 2, step, 0)

        for c in (nchunk - 2, nchunk - 1):
            scatter_wait(c, c % 2, sems_s[c % 2])

        plsc.subcore_barrier()
        plsc.subcore_barrier()

        @pl.when(sid < NS - 1)
        def _write_main():
            pltpu.sync_copy(acc_sh.at[pl.ds(rbase, r0)],
                            out_hbm.at[pl.ds(cid * n + rbase, r0)])

        @pl.when(sid == NS - 1)
        def _write_last():
            pltpu.sync_copy(
                acc_sh.at[pl.ds((NS - 1) * r0, r_last)],
                out_hbm.at[pl.ds(cid * n + (NS - 1) * r0, r_last)])

    return spmm(x, src, dst, vals, zrows)


def _tc_combine_matmul(p0, p1, W, b2d):
    n, d_in = p0.shape
    d_out = W.shape[1]
    bm = 1000

    def body(p0_ref, p1_ref, w_ref, b_ref, o_ref):
        acc = p0_ref[...] + p1_ref[...]
        o_ref[...] = (
            jnp.dot(acc, w_ref[...], preferred_element_type=jnp.float32)
            + b_ref[...])

    return pl.pallas_call(
        body,
        grid=(n // bm,),
        in_specs=[
            pl.BlockSpec((bm, d_in), lambda i: (i, 0)),
            pl.BlockSpec((bm, d_in), lambda i: (i, 0)),
            pl.BlockSpec((d_in, d_out), lambda i: (0, 0)),
            pl.BlockSpec((1, d_out), lambda i: (0, 0)),
        ],
        out_specs=pl.BlockSpec((bm, d_out), lambda i: (i, 0)),
        out_shape=jax.ShapeDtypeStruct((n, d_out), jnp.float32),
    )(p0, p1, W, b2d)


def kernel(x, edge_index, edge_vals, W, b):
    n, d = x.shape
    e = edge_vals.shape[0]
    assert e % (NC * NS * CHUNK) == 0 and n % NS == 0 and d % L == 0

    src = edge_index[1]
    dst = edge_index[0].reshape(-1, 5, CHUNK)
    r_last = n - (NS - 1) * ((n // NS) // 8 * 8)
    zrows = jnp.zeros((r_last, d), jnp.float32)

    partials = _sc_spmm(x, src, dst, edge_vals, zrows)
    return _tc_combine_matmul(partials[:n], partials[n:], W,
                              b.reshape(1, -1))
